# baseline (speedup 1.0000x reference)
import jax
import jax.numpy as jnp
from jax.experimental import pallas as pl
from jax.experimental.pallas import tpu as pltpu


def _pos_body(col_hbm, row_hbm, out_ref, col_v, row_v, sem):
    h, w = out_ref.shape[1], out_ref.shape[2]
    d = col_v.shape[1]

    @pl.when(pl.program_id(0) == 0)
    def _stage():
        cp = pltpu.make_async_copy(col_hbm, col_v, sem)
        cp.start()
        cp.wait()
        cp2 = pltpu.make_async_copy(row_hbm, row_v, sem)
        cp2.start()
        cp2.wait()

    col_img = jnp.broadcast_to(col_v[...][None, :, :], (h, w, d))
    row_img = jnp.broadcast_to(row_v[...][:, None, :], (h, w, d))
    out_ref[...] = jnp.concatenate([col_img, row_img], axis=-1)[None]


def kernel(x, row_embed, col_embed):
    b = x.shape[0]
    h, w = x.shape[-2], x.shape[-1]
    d = col_embed.shape[1]
    out = pl.pallas_call(
        _pos_body,
        grid=(b,),
        in_specs=[
            pl.BlockSpec(memory_space=pltpu.HBM),
            pl.BlockSpec(memory_space=pltpu.HBM),
        ],
        out_specs=pl.BlockSpec((1, h, w, 2 * d), lambda i: (i, 0, 0, 0)),
        out_shape=jax.ShapeDtypeStruct((b, h, w, 2 * d), jnp.float32),
        scratch_shapes=[
            pltpu.VMEM((w, d), jnp.float32),
            pltpu.VMEM((h, d), jnp.float32),
            pltpu.SemaphoreType.DMA,
        ],
    )(col_embed[:w], row_embed[:h])
    return jnp.transpose(out, (0, 3, 1, 2))


# speedup vs baseline: 1.7731x; 1.7731x over previous
import jax
import jax.numpy as jnp
from jax.experimental import pallas as pl


def _pos_body(col_ref, row_ref, out_ref):
    h, w = out_ref.shape[1], out_ref.shape[2]
    d = col_ref.shape[1]
    col_img = jnp.broadcast_to(col_ref[...][None, :, :], (h, w, d))
    row_img = jnp.broadcast_to(row_ref[...][:, None, :], (h, w, d))
    out_ref[...] = jnp.concatenate([col_img, row_img], axis=-1)[None]


def kernel(x, row_embed, col_embed):
    b = x.shape[0]
    h, w = x.shape[-2], x.shape[-1]
    d = col_embed.shape[1]
    out = pl.pallas_call(
        _pos_body,
        grid=(b,),
        in_specs=[
            pl.BlockSpec((w, d), lambda i: (0, 0)),
            pl.BlockSpec((h, d), lambda i: (0, 0)),
        ],
        out_specs=pl.BlockSpec((1, h, w, 2 * d), lambda i: (i, 0, 0, 0)),
        out_shape=jax.ShapeDtypeStruct((b, h, w, 2 * d), jnp.float32),
    )(col_embed, row_embed)
    return jnp.transpose(out, (0, 3, 1, 2))


# speedup vs baseline: 1.9207x; 1.0832x over previous
import jax
import jax.numpy as jnp
from jax.experimental import pallas as pl
from jax.experimental.pallas import tpu as pltpu


def _pos_body(col_ref, row_ref, out_hbm, slab, sem):
    h, w = slab.shape[0], slab.shape[1]
    d = col_ref.shape[1]
    b = out_hbm.shape[0]
    col_img = jnp.broadcast_to(col_ref[...][None, :, :], (h, w, d))
    row_img = jnp.broadcast_to(row_ref[...][:, None, :], (h, w, d))
    slab[...] = jnp.concatenate([col_img, row_img], axis=-1)
    copies = [
        pltpu.make_async_copy(slab, out_hbm.at[i], sem) for i in range(b)
    ]
    for cp in copies:
        cp.start()
    for cp in copies:
        cp.wait()


def kernel(x, row_embed, col_embed):
    b = x.shape[0]
    h, w = x.shape[-2], x.shape[-1]
    d = col_embed.shape[1]
    out = pl.pallas_call(
        _pos_body,
        grid=(1,),
        in_specs=[
            pl.BlockSpec((w, d), lambda i: (0, 0)),
            pl.BlockSpec((h, d), lambda i: (0, 0)),
        ],
        out_specs=pl.BlockSpec(memory_space=pltpu.HBM),
        out_shape=jax.ShapeDtypeStruct((b, h, w, 2 * d), jnp.float32),
        scratch_shapes=[
            pltpu.VMEM((h, w, 2 * d), jnp.float32),
            pltpu.SemaphoreType.DMA,
        ],
    )(col_embed, row_embed)
    return jnp.transpose(out, (0, 3, 1, 2))
